# Initial kernel scaffold; baseline (speedup 1.0000x reference)
import jax
import jax.numpy as jnp
from jax.experimental import pallas as pl


def kernel(location_x, location_table, user_table, timeslot_table):
    raise NotImplementedError("write your pallas kernel here")


# speedup vs baseline: 3.4008x; 3.4008x over previous
import functools

import jax
import jax.numpy as jnp
from jax import lax
from jax.experimental import pallas as pl
from jax.experimental.pallas import tpu as pltpu
from jax.experimental.pallas import tpu_sc as plsc

NUM_LOCATIONS = 100000
NUM_USERS = 100000
DIM = 64
BATCH = 4096
HIST = 50
B = BATCH * HIST

NC, NS = 2, 16
NW = NC * NS
B_PER_W = B // NW
G_CHUNK = 800
N_GCHUNK = B_PER_W // G_CHUNK

U_PER_W = 3128
U_CHUNK = 800
N_UCHUNK = 3

_mesh = plsc.VectorSubcoreMesh(core_axis_name="c", subcore_axis_name="s")


@functools.partial(
    pl.kernel,
    mesh=_mesh,
    out_type=[
        jax.ShapeDtypeStruct((B, DIM), jnp.float32),
        jax.ShapeDtypeStruct((24, DIM), jnp.float32),
        jax.ShapeDtypeStruct((NUM_USERS, DIM), jnp.float32),
    ],
    scratch_types=[
        pltpu.VMEM((B_PER_W,), jnp.int32),
        pltpu.VMEM((G_CHUNK, DIM), jnp.float32),
        pltpu.VMEM((G_CHUNK, DIM), jnp.float32),
        pltpu.SemaphoreType.DMA,
        pltpu.SemaphoreType.DMA,
        pltpu.SemaphoreType.DMA,
        pltpu.SemaphoreType.DMA,
    ],
    compiler_params=pltpu.CompilerParams(use_tc_tiling_on_sc=False),
)
def _emb_kernel(idx_hbm, loc_tab, user_tab, ts_tab,
                loc_out, ts_out, user_out,
                idx_v, buf0, buf1, gsem0, gsem1, wsem0, wsem1):
    wid = lax.axis_index("s") * NC + lax.axis_index("c")
    gbase = wid * B_PER_W
    ubase = wid * U_PER_W

    pltpu.sync_copy(idx_hbm.at[pl.ds(gbase, B_PER_W)], idx_v)

    bufs = (buf0, buf1)
    gsems = (gsem0, gsem1)
    wsems = (wsem0, wsem1)

    jobs = [("g", c) for c in range(N_GCHUNK)] + [("u", c) for c in range(N_UCHUNK)]

    def start_read(job, buf, sem):
        kind, c = job
        if kind == "g":
            return pltpu.async_copy(
                loc_tab.at[idx_v.at[pl.ds(c * G_CHUNK, G_CHUNK)]], buf, sem)
        return pltpu.async_copy(
            user_tab.at[pl.ds(ubase + c * U_CHUNK, U_CHUNK)], buf, sem)

    def start_write(job, buf, sem):
        kind, c = job
        if kind == "g":
            return pltpu.async_copy(
                buf, loc_out.at[pl.ds(gbase + c * G_CHUNK, G_CHUNK)], sem)
        return pltpu.async_copy(
            buf, user_out.at[pl.ds(ubase + c * U_CHUNK, U_CHUNK)], sem)

    reads = [None, None]
    writes = [None, None]
    for i, job in enumerate(jobs):
        b = i & 1
        if writes[b] is not None:
            writes[b].wait()
        reads[b] = start_read(job, bufs[b], gsems[b])
        if i > 0:
            pb = 1 - b
            reads[pb].wait()
            writes[pb] = start_write(jobs[i - 1], bufs[pb], wsems[pb])
    last = (len(jobs) - 1) & 1
    reads[last].wait()
    writes[last] = start_write(jobs[-1], bufs[last], wsems[last])
    writes[0].wait()
    writes[1].wait()

    tail_base = ubase + N_UCHUNK * U_CHUNK

    @pl.when(wid < NW - 1)
    def _():
        pltpu.sync_copy(user_tab.at[pl.ds(tail_base, 728)],
                        buf0.at[pl.ds(0, 728)])
        pltpu.sync_copy(buf0.at[pl.ds(0, 728)],
                        user_out.at[pl.ds(tail_base, 728)])

    @pl.when(wid == NW - 1)
    def _():
        pltpu.sync_copy(user_tab.at[pl.ds(tail_base, 632)],
                        buf1.at[pl.ds(0, 632)])
        pltpu.sync_copy(buf1.at[pl.ds(0, 632)],
                        user_out.at[pl.ds(tail_base, 632)])

    @pl.when(wid == 0)
    def _():
        pltpu.sync_copy(ts_tab, buf1.at[pl.ds(0, 24)])
        pltpu.sync_copy(buf1.at[pl.ds(0, 24)], ts_out)


def kernel(location_x, location_table, user_table, timeslot_table):
    idx = location_x.reshape(B).astype(jnp.int32)
    loc_flat, ts, user = _emb_kernel(
        idx, location_table, user_table, timeslot_table)
    return loc_flat.reshape(BATCH, HIST, DIM), ts, user


# speedup vs baseline: 3.7353x; 1.0984x over previous
import functools

import jax
import jax.numpy as jnp
from jax import lax
from jax.experimental import pallas as pl
from jax.experimental.pallas import tpu as pltpu
from jax.experimental.pallas import tpu_sc as plsc

NUM_LOCATIONS = 100000
NUM_USERS = 100000
DIM = 64
BATCH = 4096
HIST = 50
B = BATCH * HIST

NC, NS = 2, 16
NW = NC * NS
B_PER_W = B // NW
G_CHUNK = 800
N_GCHUNK = B_PER_W // G_CHUNK

_mesh = plsc.VectorSubcoreMesh(core_axis_name="c", subcore_axis_name="s")


@functools.partial(
    pl.kernel,
    mesh=_mesh,
    out_type=[
        jax.ShapeDtypeStruct((B, DIM), jnp.float32),
        jax.ShapeDtypeStruct((24, DIM), jnp.float32),
    ],
    scratch_types=[
        pltpu.VMEM((B_PER_W,), jnp.int32),
        pltpu.VMEM((G_CHUNK, DIM), jnp.float32),
        pltpu.VMEM((G_CHUNK, DIM), jnp.float32),
        pltpu.SemaphoreType.DMA,
        pltpu.SemaphoreType.DMA,
        pltpu.SemaphoreType.DMA,
        pltpu.SemaphoreType.DMA,
    ],
    compiler_params=pltpu.CompilerParams(use_tc_tiling_on_sc=False),
)
def _gather_kernel(idx_hbm, loc_tab, ts_tab, loc_out, ts_out,
                   idx_v, buf0, buf1, gsem0, gsem1, wsem0, wsem1):
    wid = lax.axis_index("s") * NC + lax.axis_index("c")
    gbase = wid * B_PER_W

    pltpu.sync_copy(idx_hbm.at[pl.ds(gbase, B_PER_W)], idx_v)

    bufs = (buf0, buf1)
    gsems = (gsem0, gsem1)
    wsems = (wsem0, wsem1)

    reads = [None, None]
    writes = [None, None]
    for c in range(N_GCHUNK):
        b = c & 1
        if writes[b] is not None:
            writes[b].wait()
        reads[b] = pltpu.async_copy(
            loc_tab.at[idx_v.at[pl.ds(c * G_CHUNK, G_CHUNK)]], bufs[b],
            gsems[b])
        if c > 0:
            pb = 1 - b
            reads[pb].wait()
            writes[pb] = pltpu.async_copy(
                bufs[pb],
                loc_out.at[pl.ds(gbase + (c - 1) * G_CHUNK, G_CHUNK)],
                wsems[pb])
    last = (N_GCHUNK - 1) & 1
    reads[last].wait()
    writes[last] = pltpu.async_copy(
        bufs[last],
        loc_out.at[pl.ds(gbase + (N_GCHUNK - 1) * G_CHUNK, G_CHUNK)],
        wsems[last])
    writes[0].wait()
    writes[1].wait()

    @pl.when(wid == 0)
    def _():
        pltpu.sync_copy(ts_tab, buf0.at[pl.ds(0, 24)])
        pltpu.sync_copy(buf0.at[pl.ds(0, 24)], ts_out)


def _copy_body(in_ref, out_ref):
    out_ref[...] = in_ref[...]


_ROWS_PER_BLK = 4000
_user_copy = pl.pallas_call(
    _copy_body,
    grid=(NUM_USERS // _ROWS_PER_BLK,),
    in_specs=[pl.BlockSpec((_ROWS_PER_BLK, DIM), lambda i: (i, 0))],
    out_specs=pl.BlockSpec((_ROWS_PER_BLK, DIM), lambda i: (i, 0)),
    out_shape=jax.ShapeDtypeStruct((NUM_USERS, DIM), jnp.float32),
)


def kernel(location_x, location_table, user_table, timeslot_table):
    idx = location_x.reshape(B).astype(jnp.int32)
    loc_flat, ts = _gather_kernel(idx, location_table, timeslot_table)
    user = _user_copy(user_table)
    return loc_flat.reshape(BATCH, HIST, DIM), ts, user


# speedup vs baseline: 4.7580x; 1.2738x over previous
import functools

import jax
import jax.numpy as jnp
from jax import lax
from jax.experimental import pallas as pl
from jax.experimental.pallas import tpu as pltpu
from jax.experimental.pallas import tpu_sc as plsc

NUM_LOCATIONS = 100000
NUM_USERS = 100000
DIM = 64
BATCH = 4096
HIST = 50
B = BATCH * HIST

NC, NS = 2, 16
NW = NC * NS
B_PER_NW = BATCH // NW
BCH = 16
N_GCHUNK = B_PER_NW // BCH

_mesh = plsc.VectorSubcoreMesh(core_axis_name="c", subcore_axis_name="s")


def _write_chunk(buf, loc_out, b0, sem):
    return [
        pltpu.async_copy(buf.at[pl.ds(k * HIST, HIST)], loc_out.at[b0 + k],
                         sem)
        for k in range(BCH)
    ]


@functools.partial(
    pl.kernel,
    mesh=_mesh,
    out_type=[
        jax.ShapeDtypeStruct((BATCH, HIST, DIM), jnp.float32),
        jax.ShapeDtypeStruct((24, DIM), jnp.float32),
    ],
    scratch_types=[
        pltpu.VMEM((B_PER_NW * HIST,), jnp.int32),
        pltpu.VMEM((BCH * HIST, DIM), jnp.float32),
        pltpu.VMEM((BCH * HIST, DIM), jnp.float32),
        pltpu.SemaphoreType.DMA,
        pltpu.SemaphoreType.DMA,
        pltpu.SemaphoreType.DMA,
        pltpu.SemaphoreType.DMA,
    ],
    compiler_params=pltpu.CompilerParams(use_tc_tiling_on_sc=False),
)
def _gather_kernel(idx_hbm, loc_tab, ts_tab, loc_out, ts_out,
                   idx_v, buf0, buf1, gsem0, gsem1, wsem0, wsem1):
    wid = lax.axis_index("s") * NC + lax.axis_index("c")
    gbase = wid * B_PER_NW

    pltpu.sync_copy(idx_hbm.at[pl.ds(gbase * HIST, B_PER_NW * HIST)], idx_v)

    bufs = (buf0, buf1)
    gsems = (gsem0, gsem1)
    wsems = (wsem0, wsem1)

    reads = [None, None]
    writes = [None, None]
    for c in range(N_GCHUNK):
        b = c & 1
        if writes[b] is not None:
            for h in writes[b]:
                h.wait()
        reads[b] = pltpu.async_copy(
            loc_tab.at[idx_v.at[pl.ds(c * BCH * HIST, BCH * HIST)]], bufs[b],
            gsems[b])
        if c > 0:
            pb = 1 - b
            reads[pb].wait()
            writes[pb] = _write_chunk(bufs[pb], loc_out,
                                      gbase + (c - 1) * BCH, wsems[pb])
    last = (N_GCHUNK - 1) & 1
    reads[last].wait()
    writes[last] = _write_chunk(bufs[last], loc_out,
                                gbase + (N_GCHUNK - 1) * BCH, wsems[last])
    for h in writes[0]:
        h.wait()
    for h in writes[1]:
        h.wait()

    @pl.when(wid == 0)
    def _():
        pltpu.sync_copy(ts_tab, buf0.at[pl.ds(0, 24)])
        pltpu.sync_copy(buf0.at[pl.ds(0, 24)], ts_out)


def _copy_body(in_ref, out_ref):
    out_ref[...] = in_ref[...]


_COLS_PER_BLK = 6400
_user_copy_t = pl.pallas_call(
    _copy_body,
    grid=(NUM_USERS // _COLS_PER_BLK + 1,),
    in_specs=[pl.BlockSpec((DIM, _COLS_PER_BLK), lambda i: (0, i))],
    out_specs=pl.BlockSpec((DIM, _COLS_PER_BLK), lambda i: (0, i)),
    out_shape=jax.ShapeDtypeStruct((DIM, NUM_USERS), jnp.float32),
)


def kernel(location_x, location_table, user_table, timeslot_table):
    idx = location_x.reshape(B).astype(jnp.int32)
    loc, ts = _gather_kernel(idx, location_table, timeslot_table)
    user = _user_copy_t(user_table.T).T
    return loc, ts, user


# speedup vs baseline: 6.0682x; 1.2754x over previous
import functools

import jax
import jax.numpy as jnp
from jax import lax
from jax.experimental import pallas as pl
from jax.experimental.pallas import tpu as pltpu
from jax.experimental.pallas import tpu_sc as plsc

NUM_LOCATIONS = 100000
NUM_USERS = 100000
DIM = 64
BATCH = 4096
HIST = 50

NC, NS = 2, 16
NW = NC * NS
B_PER_NW = BATCH // NW

_mesh = plsc.VectorSubcoreMesh(core_axis_name="c", subcore_axis_name="s")


@functools.partial(
    pl.kernel,
    mesh=_mesh,
    out_type=[
        jax.ShapeDtypeStruct((HIST, BATCH // 2, 2 * DIM), jnp.float32),
        jax.ShapeDtypeStruct((24, DIM), jnp.float32),
    ],
    scratch_types=[
        pltpu.VMEM((HIST, B_PER_NW), jnp.int32),
        pltpu.VMEM((B_PER_NW, DIM), jnp.float32),
        pltpu.VMEM((B_PER_NW, DIM), jnp.float32),
        pltpu.SemaphoreType.DMA,
        pltpu.SemaphoreType.DMA,
        pltpu.SemaphoreType.DMA,
        pltpu.SemaphoreType.DMA,
    ],
    compiler_params=pltpu.CompilerParams(use_tc_tiling_on_sc=False),
)
def _gather_kernel(idx_hbm, loc_tab, ts_tab, loc_out, ts_out,
                   idx_v, buf0, buf1, gsem0, gsem1, wsem0, wsem1):
    wid = lax.axis_index("s") * NC + lax.axis_index("c")
    b0 = wid * B_PER_NW

    pltpu.sync_copy(idx_hbm.at[:, pl.ds(b0, B_PER_NW)], idx_v)

    bufs = (buf0, buf1)
    gsems = (gsem0, gsem1)
    wsems = (wsem0, wsem1)

    q0 = wid * (B_PER_NW // 2)
    hw = B_PER_NW // 2

    def _write(h, buf, sem):
        return (
            pltpu.async_copy(
                buf.at[pl.ds(0, hw)],
                loc_out.at[h, pl.ds(q0, hw), pl.ds(0, DIM)], sem),
            pltpu.async_copy(
                buf.at[pl.ds(hw, hw)],
                loc_out.at[h, pl.ds(q0, hw), pl.ds(DIM, DIM)], sem),
        )

    reads = [None, None]
    writes = [None, None]
    for h in range(HIST):
        b = h & 1
        if writes[b] is not None:
            writes[b][0].wait()
            writes[b][1].wait()
        reads[b] = pltpu.async_copy(
            loc_tab.at[idx_v.at[h]], bufs[b], gsems[b])
        if h > 0:
            pb = 1 - b
            reads[pb].wait()
            writes[pb] = _write(h - 1, bufs[pb], wsems[pb])
    last = (HIST - 1) & 1
    reads[last].wait()
    writes[last] = _write(HIST - 1, bufs[last], wsems[last])
    writes[0][0].wait()
    writes[0][1].wait()
    writes[1][0].wait()
    writes[1][1].wait()

    @pl.when(wid == 0)
    def _():
        pltpu.sync_copy(ts_tab, buf0.at[pl.ds(0, 24)])
        pltpu.sync_copy(buf0.at[pl.ds(0, 24)], ts_out)


def _copy_body(in_ref, out_ref):
    out_ref[...] = in_ref[...]


_COLS_PER_BLK = 6400
_user_copy_t = pl.pallas_call(
    _copy_body,
    grid=(NUM_USERS // _COLS_PER_BLK + 1,),
    in_specs=[pl.BlockSpec((DIM, _COLS_PER_BLK), lambda i: (0, i))],
    out_specs=pl.BlockSpec((DIM, _COLS_PER_BLK), lambda i: (0, i)),
    out_shape=jax.ShapeDtypeStruct((DIM, NUM_USERS), jnp.float32),
)


def _unpack_body(x_ref, y_ref):
    x = x_ref[0]
    y_ref[0] = jnp.concatenate([x[:, :DIM].T, x[:, DIM:].T], axis=1)


_unpack = pl.pallas_call(
    _unpack_body,
    grid=(HIST,),
    in_specs=[pl.BlockSpec((1, BATCH // 2, 2 * DIM), lambda h: (h, 0, 0))],
    out_specs=pl.BlockSpec((1, DIM, BATCH), lambda h: (h, 0, 0)),
    out_shape=jax.ShapeDtypeStruct((HIST, DIM, BATCH), jnp.float32),
)


def kernel(location_x, location_table, user_table, timeslot_table):
    idx_t = location_x.T.astype(jnp.int32)
    idx_p = (idx_t.reshape(HIST, 2, NW, B_PER_NW // 2)
             .transpose(0, 2, 1, 3).reshape(HIST, BATCH))
    loc_p, ts = _gather_kernel(idx_p, location_table, timeslot_table)
    loc = jnp.transpose(_unpack(loc_p), (2, 0, 1))
    user = _user_copy_t(user_table.T).T
    return loc, ts, user


# speedup vs baseline: 6.2181x; 1.0247x over previous
import functools

import jax
import jax.numpy as jnp
from jax import lax
from jax.experimental import pallas as pl
from jax.experimental.pallas import tpu as pltpu
from jax.experimental.pallas import tpu_sc as plsc

NUM_LOCATIONS = 100000
NUM_USERS = 100000
DIM = 64
BATCH = 4096
HIST = 50

NC, NS = 2, 16
NW = NC * NS
B_PER_NW = BATCH // NW

_mesh = plsc.VectorSubcoreMesh(core_axis_name="c", subcore_axis_name="s")


@functools.partial(
    pl.kernel,
    mesh=_mesh,
    out_type=[
        jax.ShapeDtypeStruct((HIST, BATCH // 2, 2 * DIM), jnp.float32),
        jax.ShapeDtypeStruct((24, DIM), jnp.float32),
    ],
    scratch_types=[
        pltpu.VMEM((HIST, B_PER_NW), jnp.int32),
        pltpu.VMEM((B_PER_NW, DIM), jnp.float32),
        pltpu.VMEM((B_PER_NW, DIM), jnp.float32),
        pltpu.VMEM((B_PER_NW, DIM), jnp.float32),
        pltpu.VMEM((B_PER_NW, DIM), jnp.float32),
        pltpu.SemaphoreType.DMA,
        pltpu.SemaphoreType.DMA,
        pltpu.SemaphoreType.DMA,
        pltpu.SemaphoreType.DMA,
        pltpu.SemaphoreType.DMA,
        pltpu.SemaphoreType.DMA,
        pltpu.SemaphoreType.DMA,
        pltpu.SemaphoreType.DMA,
    ],
    compiler_params=pltpu.CompilerParams(use_tc_tiling_on_sc=False),
)
def _gather_kernel(idx_hbm, loc_tab, ts_tab, loc_out, ts_out,
                   idx_v, buf0, buf1, buf2, buf3,
                   gsem0, gsem1, gsem2, gsem3,
                   wsem0, wsem1, wsem2, wsem3):
    wid = lax.axis_index("s") * NC + lax.axis_index("c")
    hw = B_PER_NW // 2
    q0 = wid * hw

    pltpu.sync_copy(idx_hbm.at[:, pl.ds(q0, hw)], idx_v.at[:, pl.ds(0, hw)])
    pltpu.sync_copy(idx_hbm.at[:, pl.ds(BATCH // 2 + q0, hw)],
                    idx_v.at[:, pl.ds(hw, hw)])

    bufs = (buf0, buf1, buf2, buf3)
    gsems = (gsem0, gsem1, gsem2, gsem3)
    wsems = (wsem0, wsem1, wsem2, wsem3)
    NBUF = 4

    def _write(h, buf, sem):
        return (
            pltpu.async_copy(
                buf.at[pl.ds(0, hw)],
                loc_out.at[h, pl.ds(q0, hw), pl.ds(0, DIM)], sem),
            pltpu.async_copy(
                buf.at[pl.ds(hw, hw)],
                loc_out.at[h, pl.ds(q0, hw), pl.ds(DIM, DIM)], sem),
        )

    reads = [None] * NBUF
    writes = [None] * NBUF
    for t in range(HIST + NBUF - 1):
        if t < HIST:
            b = t % NBUF
            if writes[b] is not None:
                writes[b][0].wait()
                writes[b][1].wait()
            reads[b] = pltpu.async_copy(
                loc_tab.at[idx_v.at[t]], bufs[b], gsems[b])
        hp = t - (NBUF - 1)
        if 0 <= hp < HIST:
            pb = hp % NBUF
            reads[pb].wait()
            writes[pb] = _write(hp, bufs[pb], wsems[pb])
    for k in range(NBUF):
        if writes[k] is not None:
            writes[k][0].wait()
            writes[k][1].wait()

    @pl.when(wid == 0)
    def _():
        pltpu.sync_copy(ts_tab, buf0.at[pl.ds(0, 24)])
        pltpu.sync_copy(buf0.at[pl.ds(0, 24)], ts_out)


def _copy_body(in_ref, out_ref):
    out_ref[...] = in_ref[...]


_COLS_PER_BLK = 6400
_user_copy_t = pl.pallas_call(
    _copy_body,
    grid=(NUM_USERS // _COLS_PER_BLK + 1,),
    in_specs=[pl.BlockSpec((DIM, _COLS_PER_BLK), lambda i: (0, i))],
    out_specs=pl.BlockSpec((DIM, _COLS_PER_BLK), lambda i: (0, i)),
    out_shape=jax.ShapeDtypeStruct((DIM, NUM_USERS), jnp.float32),
)


def _unpack_body(x_ref, y_ref):
    x = x_ref[0]
    y_ref[0] = jnp.concatenate([x[:, :DIM].T, x[:, DIM:].T], axis=1)


_unpack = pl.pallas_call(
    _unpack_body,
    grid=(HIST,),
    in_specs=[pl.BlockSpec((1, BATCH // 2, 2 * DIM), lambda h: (h, 0, 0))],
    out_specs=pl.BlockSpec((1, DIM, BATCH), lambda h: (h, 0, 0)),
    out_shape=jax.ShapeDtypeStruct((HIST, DIM, BATCH), jnp.float32),
)


def kernel(location_x, location_table, user_table, timeslot_table):
    idx_t = location_x.T.astype(jnp.int32)
    loc_p, ts = _gather_kernel(idx_t, location_table, timeslot_table)
    loc = jnp.transpose(_unpack(loc_p), (2, 0, 1))
    user = _user_copy_t(user_table.T).T
    return loc, ts, user


# speedup vs baseline: 6.2242x; 1.0010x over previous
import functools

import jax
import jax.numpy as jnp
from jax import lax
from jax.experimental import pallas as pl
from jax.experimental.pallas import tpu as pltpu
from jax.experimental.pallas import tpu_sc as plsc

NUM_LOCATIONS = 100000
NUM_USERS = 100000
DIM = 64
BATCH = 4096
HIST = 50

NC, NS = 2, 16
NW = NC * NS
B_PER_NW = BATCH // NW

_mesh = plsc.VectorSubcoreMesh(core_axis_name="c", subcore_axis_name="s")


@functools.partial(
    pl.kernel,
    mesh=_mesh,
    out_type=[
        jax.ShapeDtypeStruct((HIST, BATCH // 2, 2 * DIM), jnp.float32),
        jax.ShapeDtypeStruct((24, DIM), jnp.float32),
    ],
    scratch_types=[
        pltpu.VMEM((HIST, B_PER_NW), jnp.int32),
        pltpu.VMEM((B_PER_NW, DIM), jnp.float32),
        pltpu.VMEM((B_PER_NW, DIM), jnp.float32),
        pltpu.VMEM((B_PER_NW, DIM), jnp.float32),
        pltpu.VMEM((B_PER_NW, DIM), jnp.float32),
        pltpu.SemaphoreType.DMA,
        pltpu.SemaphoreType.DMA,
        pltpu.SemaphoreType.DMA,
        pltpu.SemaphoreType.DMA,
        pltpu.SemaphoreType.DMA,
        pltpu.SemaphoreType.DMA,
        pltpu.SemaphoreType.DMA,
        pltpu.SemaphoreType.DMA,
        pltpu.SemaphoreType.DMA,
    ],
    compiler_params=pltpu.CompilerParams(use_tc_tiling_on_sc=False),
)
def _gather_kernel(idx_hbm, loc_tab, ts_tab, loc_out, ts_out,
                   idx_v, buf0, buf1, buf2, buf3,
                   gsem0, gsem1, gsem2, gsem3,
                   wsem0, wsem1, wsem2, wsem3, isem):
    wid = lax.axis_index("s") * NC + lax.axis_index("c")
    hw = B_PER_NW // 2
    q0 = wid * hw

    ih = []
    for h in range(HIST):
        ih.append(pltpu.async_copy(
            idx_hbm.at[pl.ds(h * BATCH + q0, hw)],
            idx_v.at[h, pl.ds(0, hw)], isem))
        ih.append(pltpu.async_copy(
            idx_hbm.at[pl.ds(h * BATCH + BATCH // 2 + q0, hw)],
            idx_v.at[h, pl.ds(hw, hw)], isem))
    for hnd in ih:
        hnd.wait()

    bufs = (buf0, buf1, buf2, buf3)
    gsems = (gsem0, gsem1, gsem2, gsem3)
    wsems = (wsem0, wsem1, wsem2, wsem3)
    NBUF = 4

    def _write(h, buf, sem):
        return (
            pltpu.async_copy(
                buf.at[pl.ds(0, hw)],
                loc_out.at[h, pl.ds(q0, hw), pl.ds(0, DIM)], sem),
            pltpu.async_copy(
                buf.at[pl.ds(hw, hw)],
                loc_out.at[h, pl.ds(q0, hw), pl.ds(DIM, DIM)], sem),
        )

    reads = [None] * NBUF
    writes = [None] * NBUF
    for t in range(HIST + NBUF - 1):
        if t < HIST:
            b = t % NBUF
            if writes[b] is not None:
                writes[b][0].wait()
                writes[b][1].wait()
            reads[b] = pltpu.async_copy(
                loc_tab.at[idx_v.at[t]], bufs[b], gsems[b])
        hp = t - (NBUF - 1)
        if 0 <= hp < HIST:
            pb = hp % NBUF
            reads[pb].wait()
            writes[pb] = _write(hp, bufs[pb], wsems[pb])
    for k in range(NBUF):
        if writes[k] is not None:
            writes[k][0].wait()
            writes[k][1].wait()

    @pl.when(wid == 0)
    def _():
        pltpu.sync_copy(ts_tab, buf0.at[pl.ds(0, 24)])
        pltpu.sync_copy(buf0.at[pl.ds(0, 24)], ts_out)


def _copy_body(in_ref, out_ref):
    out_ref[...] = in_ref[...]


_COLS_PER_BLK = 6400
_user_copy_t = pl.pallas_call(
    _copy_body,
    grid=(NUM_USERS // _COLS_PER_BLK + 1,),
    in_specs=[pl.BlockSpec((DIM, _COLS_PER_BLK), lambda i: (0, i))],
    out_specs=pl.BlockSpec((DIM, _COLS_PER_BLK), lambda i: (0, i)),
    out_shape=jax.ShapeDtypeStruct((DIM, NUM_USERS), jnp.float32),
)


def _unpack_body(x_ref, y_ref):
    x = x_ref[0]
    y_ref[0] = jnp.concatenate([x[:, :DIM].T, x[:, DIM:].T], axis=1)


_unpack = pl.pallas_call(
    _unpack_body,
    grid=(HIST,),
    in_specs=[pl.BlockSpec((1, BATCH // 2, 2 * DIM), lambda h: (h, 0, 0))],
    out_specs=pl.BlockSpec((1, DIM, BATCH), lambda h: (h, 0, 0)),
    out_shape=jax.ShapeDtypeStruct((HIST, DIM, BATCH), jnp.float32),
)


def kernel(location_x, location_table, user_table, timeslot_table):
    idx_t = location_x.T.reshape(BATCH * HIST).astype(jnp.int32)
    loc_p, ts = _gather_kernel(idx_t, location_table, timeslot_table)
    loc = jnp.transpose(_unpack(loc_p), (2, 0, 1))
    user = _user_copy_t(user_table.T).T
    return loc, ts, user
